# baseline (speedup 1.0000x reference)
import functools

import jax
import jax.numpy as jnp
from jax import lax
from jax.experimental import pallas as pl
from jax.experimental.pallas import tpu as pltpu
from jax.experimental.pallas import tpu_sc as plsc

NC = 2
NS = 16
NW = NC * NS
K = 64
SCH = 8


def _zero_vmem_2d(ref, rows, cols):
    zv = jnp.zeros((16,), jnp.float32)

    def body(i, _):
        for k in range(cols // 16):
            ref[i, pl.ds(16 * k, 16)] = zv
        return 0

    lax.fori_loop(0, rows, body, 0)


def _zero_vmem_1d(ref, n):
    zv = jnp.zeros((16,), jnp.float32)

    def body(i, _):
        ref[pl.ds(16 * i, 16)] = zv
        return 0

    lax.fori_loop(0, n // 16, body, 0)


def _make_deg_kernel(n_pad, supers):
    per_tile = n_pad // NS
    mesh = plsc.VectorSubcoreMesh(core_axis_name="c", subcore_axis_name="s")

    @functools.partial(
        pl.kernel,
        out_type=jax.ShapeDtypeStruct((NC, n_pad), jnp.float32),
        mesh=mesh,
        scratch_types=[
            pltpu.VMEM((2, SCH, K), jnp.int32),
            pltpu.VMEM((K,), jnp.float32),
            pltpu.VMEM((per_tile,), jnp.float32),
            pltpu.VMEM_SHARED((n_pad,), jnp.float32),
            pltpu.SemaphoreType.DMA,
        ],
    )
    def deg_kernel(rows_hbm, deg_out, rowb, ones_v, zv, deg_sh, sem_i):
        c = lax.axis_index("c")
        s = lax.axis_index("s")
        wid = c * NS + s
        ov = jnp.ones((16,), jnp.float32)
        for k in range(K // 16):
            ones_v[pl.ds(16 * k, 16)] = ov
        _zero_vmem_1d(zv, per_tile)
        pltpu.sync_copy(zv, deg_sh.at[pl.ds(per_tile * s, per_tile)])
        plsc.subcore_barrier()

        pltpu.sync_copy(rows_hbm.at[wid, 0], rowb.at[0])

        def super_body(u, _):
            p = lax.rem(u, 2)

            @pl.when(u > 0)
            def _():
                pltpu.make_async_copy(
                    rows_hbm.at[wid, u], rowb.at[p], sem_i).wait()

            @pl.when(u + 1 < supers)
            def _():
                pltpu.async_copy(rows_hbm.at[wid, u + 1], rowb.at[1 - p],
                                 sem_i)

            for j in range(SCH):
                pltpu.sync_copy(ones_v, deg_sh.at[rowb.at[p, j]], add=True)
            return 0

        lax.fori_loop(0, supers, super_body, 0)
        plsc.subcore_barrier()
        pltpu.sync_copy(deg_sh.at[pl.ds(per_tile * s, per_tile)],
                        deg_out.at[c, pl.ds(per_tile * s, per_tile)])

    return deg_kernel


def _make_agg_kernel(n, n_pad, d, supers):
    per_tile = n_pad // NS
    mesh = plsc.VectorSubcoreMesh(core_axis_name="c", subcore_axis_name="s")

    @functools.partial(
        pl.kernel,
        out_type=jax.ShapeDtypeStruct((NC, n_pad, d), jnp.float32),
        mesh=mesh,
        scratch_types=[
            pltpu.VMEM((2, SCH, K), jnp.int32),
            pltpu.VMEM((2, SCH, K), jnp.int32),
            pltpu.VMEM((4, K, d), jnp.float32),
            pltpu.VMEM_SHARED((n_pad, d), jnp.float32),
            pltpu.SemaphoreType.DMA,
            pltpu.SemaphoreType.DMA,
            pltpu.SemaphoreType.DMA,
            pltpu.SemaphoreType.DMA,
            pltpu.SemaphoreType.DMA,
            pltpu.SemaphoreType.DMA,
            pltpu.SemaphoreType.DMA,
            pltpu.SemaphoreType.DMA,
            pltpu.SemaphoreType.DMA,
        ],
    )
    def agg_kernel(xs_hbm, cols_hbm, rows_hbm, acc_out,
                   colb, rowb, ring, acc_sh,
                   sem_i, g0, g1, g2, g3, s0, s1, s2, s3):
        c = lax.axis_index("c")
        s = lax.axis_index("s")
        wid = c * NS + s
        gsem = (g0, g1, g2, g3)
        ssem = (s0, s1, s2, s3)

        _zero_vmem_2d(ring.at[0], K, d)
        for t in range(per_tile // K):
            pltpu.sync_copy(ring.at[0],
                            acc_sh.at[pl.ds(per_tile * s + K * t, K)])
        plsc.subcore_barrier()

        pltpu.sync_copy(cols_hbm.at[wid, 0], colb.at[0])
        pltpu.sync_copy(rows_hbm.at[wid, 0], rowb.at[0])

        def step(u, p, jj, first):
            b = jj % 4
            bq = (jj + 2) % 4
            if (not first) or jj >= 4:
                pltpu.make_async_copy(
                    ring.at[b], acc_sh.at[rowb.at[p, jj]], ssem[b]).wait()
            pltpu.async_copy(xs_hbm.at[colb.at[p, jj]], ring.at[b], gsem[b])
            if (not first) or jj >= 2:
                qjj = (jj - 2) % SCH
                qp = p if jj >= 2 else 1 - p
                pltpu.make_async_copy(
                    xs_hbm.at[colb.at[qp, qjj]], ring.at[bq],
                    gsem[bq]).wait()
                pltpu.async_copy(ring.at[bq], acc_sh.at[rowb.at[qp, qjj]],
                                 ssem[bq], add=True)
            if jj == 4:
                if first:
                    if supers > 1:
                        pltpu.async_copy(cols_hbm.at[wid, 1], colb.at[1],
                                         sem_i)
                        pltpu.async_copy(rows_hbm.at[wid, 1], rowb.at[1],
                                         sem_i)
                else:
                    @pl.when(u + 1 < supers)
                    def _():
                        pltpu.async_copy(cols_hbm.at[wid, u + 1],
                                         colb.at[1 - p], sem_i)
                        pltpu.async_copy(rows_hbm.at[wid, u + 1],
                                         rowb.at[1 - p], sem_i)

        for jj in range(SCH):
            step(0, 0, jj, True)

        def super_body(u, _):
            p = lax.rem(u, 2)
            pltpu.make_async_copy(cols_hbm.at[wid, u], colb.at[p],
                                  sem_i).wait()
            pltpu.make_async_copy(rows_hbm.at[wid, u], rowb.at[p],
                                  sem_i).wait()
            for jj in range(SCH):
                step(u, p, jj, False)
            return 0

        lax.fori_loop(1, supers, super_body, 0)

        p_last = (supers - 1) % 2
        for qjj in (SCH - 2, SCH - 1):
            bq = qjj % 4
            pltpu.make_async_copy(
                xs_hbm.at[colb.at[p_last, qjj]], ring.at[bq],
                gsem[bq]).wait()
            pltpu.async_copy(ring.at[bq], acc_sh.at[rowb.at[p_last, qjj]],
                             ssem[bq], add=True)
        for b in range(4):
            pltpu.make_async_copy(
                ring.at[b], acc_sh.at[rowb.at[0, 0]], ssem[b]).wait()

        plsc.subcore_barrier()
        pltpu.sync_copy(acc_sh.at[pl.ds(per_tile * s, per_tile)],
                        acc_out.at[c, pl.ds(per_tile * s, per_tile)])

    return agg_kernel


def _scale_body(blk, deg_ref, x_ref, xs_ref):
    i = pl.program_id(0)
    dblk = deg_ref[:, pl.ds(i * blk, blk)]
    deg = dblk[0, :] + dblk[1, :] + 1.0
    dis = lax.rsqrt(deg)
    xs_ref[...] = x_ref[...] * dis[:, None]


def _epilogue_body(blk, deg_ref, acc_ref, xs_ref, w_ref, b_ref, out_ref):
    i = pl.program_id(0)
    dblk = deg_ref[:, pl.ds(i * blk, blk)]
    deg = dblk[0, :] + dblk[1, :] + 1.0
    dis = lax.rsqrt(deg)
    v = acc_ref[0] + acc_ref[1] + xs_ref[...]
    v = v * dis[:, None]
    out_ref[...] = (
        jnp.dot(v, w_ref[...], preferred_element_type=jnp.float32)
        + b_ref[...]
    )


def kernel(x, edge_index, weight, biases):
    n, d = x.shape
    e = edge_index.shape[1]
    dout = weight.shape[1]

    sup_e = SCH * K
    per_tile_e = -(-e // NW)
    per_tile_e = ((per_tile_e + sup_e - 1) // sup_e) * sup_e
    e_pad = NW * per_tile_e
    n_pad = ((n + NS * K - 1) // (NS * K)) * (NS * K)

    pad = e_pad - e
    pad_rows = n + (jnp.arange(pad, dtype=jnp.int32) % (n_pad - n))
    pad_cols = jnp.arange(pad, dtype=jnp.int32) % n
    rows = jnp.concatenate([edge_index[0], pad_rows])
    cols = jnp.concatenate([edge_index[1], pad_cols])

    supers = per_tile_e // sup_e
    rows4 = rows.reshape(NW, supers, SCH, K)
    cols4 = cols.reshape(NW, supers, SCH, K)
    deg2 = _make_deg_kernel(n_pad, supers)(rows4)

    blk = 512
    grid = -(-n // blk)
    xs = pl.pallas_call(
        functools.partial(_scale_body, blk),
        grid=(grid,),
        in_specs=[
            pl.BlockSpec((NC, n_pad), lambda i: (0, 0)),
            pl.BlockSpec((blk, d), lambda i: (i, 0)),
        ],
        out_specs=pl.BlockSpec((blk, d), lambda i: (i, 0)),
        out_shape=jax.ShapeDtypeStruct((n, d), jnp.float32),
    )(deg2, x)

    acc = _make_agg_kernel(n, n_pad, d, supers)(xs, cols4, rows4)

    out = pl.pallas_call(
        functools.partial(_epilogue_body, blk),
        grid=(grid,),
        in_specs=[
            pl.BlockSpec((NC, n_pad), lambda i: (0, 0)),
            pl.BlockSpec((NC, blk, d), lambda i: (0, i, 0)),
            pl.BlockSpec((blk, d), lambda i: (i, 0)),
            pl.BlockSpec((d, dout), lambda i: (0, 0)),
            pl.BlockSpec((1, dout), lambda i: (0, 0)),
        ],
        out_specs=pl.BlockSpec((blk, dout), lambda i: (i, 0)),
        out_shape=jax.ShapeDtypeStruct((n, dout), jnp.float32),
    )(deg2, acc, xs, weight, biases[None, :])
    return out


# speedup vs baseline: 1.0518x; 1.0518x over previous
import functools

import jax
import jax.numpy as jnp
from jax import lax
from jax.experimental import pallas as pl
from jax.experimental.pallas import tpu as pltpu
from jax.experimental.pallas import tpu_sc as plsc

NC = 2
NS = 16
NW = NC * NS
K = 64
SCH = 8


def _zero_vmem_2d(ref, rows, cols):
    zv = jnp.zeros((16,), jnp.float32)

    def body(i, _):
        for k in range(cols // 16):
            ref[i, pl.ds(16 * k, 16)] = zv
        return 0

    lax.fori_loop(0, rows, body, 0)


def _zero_vmem_1d(ref, n):
    zv = jnp.zeros((16,), jnp.float32)

    def body(i, _):
        ref[pl.ds(16 * i, 16)] = zv
        return 0

    lax.fori_loop(0, n // 16, body, 0)


DEG_K = 128
DEG_SCH = 16


def _make_deg_kernel(n_pad, supers):
    per_tile = n_pad // NS
    mesh = plsc.VectorSubcoreMesh(core_axis_name="c", subcore_axis_name="s")

    @functools.partial(
        pl.kernel,
        out_type=jax.ShapeDtypeStruct((NC, n_pad), jnp.float32),
        mesh=mesh,
        scratch_types=[
            pltpu.VMEM((2, DEG_SCH, DEG_K), jnp.int32),
            pltpu.VMEM((DEG_K,), jnp.float32),
            pltpu.VMEM((per_tile,), jnp.float32),
            pltpu.VMEM_SHARED((n_pad,), jnp.float32),
            pltpu.SemaphoreType.DMA,
            pltpu.SemaphoreType.DMA,
        ],
    )
    def deg_kernel(rows_hbm, deg_out, rowb, ones_v, zv, deg_sh, sem_i,
                   sem_a):
        c = lax.axis_index("c")
        s = lax.axis_index("s")
        wid = c * NS + s
        ov = jnp.ones((16,), jnp.float32)
        for k in range(DEG_K // 16):
            ones_v[pl.ds(16 * k, 16)] = ov
        _zero_vmem_1d(zv, per_tile)
        pltpu.sync_copy(zv, deg_sh.at[pl.ds(per_tile * s, per_tile)])
        plsc.subcore_barrier()

        pltpu.sync_copy(rows_hbm.at[wid, 0], rowb.at[0])

        def super_body(u, _):
            p = lax.rem(u, 2)

            @pl.when(u > 0)
            def _():
                pltpu.make_async_copy(
                    rows_hbm.at[wid, u], rowb.at[p], sem_i).wait()

            @pl.when(u + 1 < supers)
            def _():
                pltpu.async_copy(rows_hbm.at[wid, u + 1], rowb.at[1 - p],
                                 sem_i)

            for j in range(DEG_SCH):
                pltpu.async_copy(ones_v, deg_sh.at[rowb.at[p, j]], sem_a,
                                 add=True)
            for j in range(DEG_SCH):
                pltpu.make_async_copy(ones_v, deg_sh.at[rowb.at[p, j]],
                                      sem_a).wait()
            return 0

        lax.fori_loop(0, supers, super_body, 0)
        plsc.subcore_barrier()
        pltpu.sync_copy(deg_sh.at[pl.ds(per_tile * s, per_tile)],
                        deg_out.at[c, pl.ds(per_tile * s, per_tile)])

    return deg_kernel


def _make_agg_kernel(n, n_pad, d, supers):
    per_tile = n_pad // NS
    mesh = plsc.VectorSubcoreMesh(core_axis_name="c", subcore_axis_name="s")

    @functools.partial(
        pl.kernel,
        out_type=jax.ShapeDtypeStruct((NC, n_pad, d), jnp.float32),
        mesh=mesh,
        scratch_types=[
            pltpu.VMEM((2, SCH, K), jnp.int32),
            pltpu.VMEM((2, SCH, K), jnp.int32),
            pltpu.VMEM((4, K, d), jnp.float32),
            pltpu.VMEM_SHARED((n_pad, d), jnp.float32),
            pltpu.SemaphoreType.DMA,
            pltpu.SemaphoreType.DMA,
            pltpu.SemaphoreType.DMA,
            pltpu.SemaphoreType.DMA,
            pltpu.SemaphoreType.DMA,
            pltpu.SemaphoreType.DMA,
            pltpu.SemaphoreType.DMA,
            pltpu.SemaphoreType.DMA,
            pltpu.SemaphoreType.DMA,
        ],
    )
    def agg_kernel(xs_hbm, cols_hbm, rows_hbm, acc_out,
                   colb, rowb, ring, acc_sh,
                   sem_i, g0, g1, g2, g3, s0, s1, s2, s3):
        c = lax.axis_index("c")
        s = lax.axis_index("s")
        wid = c * NS + s
        gsem = (g0, g1, g2, g3)
        ssem = (s0, s1, s2, s3)

        _zero_vmem_2d(ring.at[0], K, d)
        for t in range(per_tile // K):
            pltpu.sync_copy(ring.at[0],
                            acc_sh.at[pl.ds(per_tile * s + K * t, K)])
        plsc.subcore_barrier()

        pltpu.sync_copy(cols_hbm.at[wid, 0], colb.at[0])
        pltpu.sync_copy(rows_hbm.at[wid, 0], rowb.at[0])

        def step(u, p, jj, first):
            b = jj % 4
            bq = (jj + 2) % 4
            if (not first) or jj >= 4:
                pltpu.make_async_copy(
                    ring.at[b], acc_sh.at[rowb.at[p, jj]], ssem[b]).wait()
            pltpu.async_copy(xs_hbm.at[colb.at[p, jj]], ring.at[b], gsem[b])
            if (not first) or jj >= 2:
                qjj = (jj - 2) % SCH
                qp = p if jj >= 2 else 1 - p
                pltpu.make_async_copy(
                    xs_hbm.at[colb.at[qp, qjj]], ring.at[bq],
                    gsem[bq]).wait()
                pltpu.async_copy(ring.at[bq], acc_sh.at[rowb.at[qp, qjj]],
                                 ssem[bq], add=True)
            if jj == 4:
                if first:
                    if supers > 1:
                        pltpu.async_copy(cols_hbm.at[wid, 1], colb.at[1],
                                         sem_i)
                        pltpu.async_copy(rows_hbm.at[wid, 1], rowb.at[1],
                                         sem_i)
                else:
                    @pl.when(u + 1 < supers)
                    def _():
                        pltpu.async_copy(cols_hbm.at[wid, u + 1],
                                         colb.at[1 - p], sem_i)
                        pltpu.async_copy(rows_hbm.at[wid, u + 1],
                                         rowb.at[1 - p], sem_i)

        for jj in range(SCH):
            step(0, 0, jj, True)

        def super_body(u, _):
            p = lax.rem(u, 2)
            pltpu.make_async_copy(cols_hbm.at[wid, u], colb.at[p],
                                  sem_i).wait()
            pltpu.make_async_copy(rows_hbm.at[wid, u], rowb.at[p],
                                  sem_i).wait()
            for jj in range(SCH):
                step(u, p, jj, False)
            return 0

        lax.fori_loop(1, supers, super_body, 0)

        p_last = (supers - 1) % 2
        for qjj in (SCH - 2, SCH - 1):
            bq = qjj % 4
            pltpu.make_async_copy(
                xs_hbm.at[colb.at[p_last, qjj]], ring.at[bq],
                gsem[bq]).wait()
            pltpu.async_copy(ring.at[bq], acc_sh.at[rowb.at[p_last, qjj]],
                             ssem[bq], add=True)
        for b in range(4):
            pltpu.make_async_copy(
                ring.at[b], acc_sh.at[rowb.at[0, 0]], ssem[b]).wait()

        plsc.subcore_barrier()
        pltpu.sync_copy(acc_sh.at[pl.ds(per_tile * s, per_tile)],
                        acc_out.at[c, pl.ds(per_tile * s, per_tile)])

    return agg_kernel


def _scale_body(blk, deg_ref, x_ref, xs_ref):
    i = pl.program_id(0)
    dblk = deg_ref[:, pl.ds(i * blk, blk)]
    deg = dblk[0, :] + dblk[1, :] + 1.0
    dis = lax.rsqrt(deg)
    xs_ref[...] = x_ref[...] * dis[:, None]


def _epilogue_body(blk, deg_ref, acc_ref, xs_ref, w_ref, b_ref, out_ref):
    i = pl.program_id(0)
    dblk = deg_ref[:, pl.ds(i * blk, blk)]
    deg = dblk[0, :] + dblk[1, :] + 1.0
    dis = lax.rsqrt(deg)
    v = acc_ref[0] + acc_ref[1] + xs_ref[...]
    v = v * dis[:, None]
    out_ref[...] = (
        jnp.dot(v, w_ref[...], preferred_element_type=jnp.float32)
        + b_ref[...]
    )


def kernel(x, edge_index, weight, biases):
    n, d = x.shape
    e = edge_index.shape[1]
    dout = weight.shape[1]

    sup_e = SCH * K
    per_tile_e = -(-e // NW)
    per_tile_e = ((per_tile_e + sup_e - 1) // sup_e) * sup_e
    e_pad = NW * per_tile_e
    n_pad = ((n + NS * K - 1) // (NS * K)) * (NS * K)

    pad = e_pad - e
    pad_rows = n + (jnp.arange(pad, dtype=jnp.int32) % (n_pad - n))
    pad_cols = jnp.arange(pad, dtype=jnp.int32) % n
    rows = jnp.concatenate([edge_index[0], pad_rows])
    cols = jnp.concatenate([edge_index[1], pad_cols])

    supers = per_tile_e // sup_e
    rows4 = rows.reshape(NW, supers, SCH, K)
    cols4 = cols.reshape(NW, supers, SCH, K)
    deg_sup = per_tile_e // (DEG_SCH * DEG_K)
    deg2 = _make_deg_kernel(n_pad, deg_sup)(
        rows.reshape(NW, deg_sup, DEG_SCH, DEG_K))

    blk = 512
    grid = -(-n // blk)
    xs = pl.pallas_call(
        functools.partial(_scale_body, blk),
        grid=(grid,),
        in_specs=[
            pl.BlockSpec((NC, n_pad), lambda i: (0, 0)),
            pl.BlockSpec((blk, d), lambda i: (i, 0)),
        ],
        out_specs=pl.BlockSpec((blk, d), lambda i: (i, 0)),
        out_shape=jax.ShapeDtypeStruct((n, d), jnp.float32),
    )(deg2, x)

    acc = _make_agg_kernel(n, n_pad, d, supers)(xs, cols4, rows4)

    out = pl.pallas_call(
        functools.partial(_epilogue_body, blk),
        grid=(grid,),
        in_specs=[
            pl.BlockSpec((NC, n_pad), lambda i: (0, 0)),
            pl.BlockSpec((NC, blk, d), lambda i: (0, i, 0)),
            pl.BlockSpec((blk, d), lambda i: (i, 0)),
            pl.BlockSpec((d, dout), lambda i: (0, 0)),
            pl.BlockSpec((1, dout), lambda i: (0, 0)),
        ],
        out_specs=pl.BlockSpec((blk, dout), lambda i: (i, 0)),
        out_shape=jax.ShapeDtypeStruct((n, dout), jnp.float32),
    )(deg2, acc, xs, weight, biases[None, :])
    return out
